# baseline (speedup 1.0000x reference)
import jax
import jax.numpy as jnp
from jax.experimental import pallas as pl

N_GROUPS = 9
N_PER_GROUP = 131072
C_IN = 64
C_OUT = 64
BLK = 2048
NB = N_PER_GROUP // BLK


def _affine_kernel(x_ref, w_ref, b_ref, o_ref):
    x = x_ref[0]
    w = w_ref[0]
    b = b_ref[0, 0]
    y = jax.lax.dot_general(
        x, w, (((1,), (1,)), ((), ())), preferred_element_type=jnp.float32
    )
    o_ref[...] = y + b[None, :]


def kernel(inputs, weights, bias):
    bias3 = bias.reshape(N_GROUPS, 1, C_OUT)
    out = pl.pallas_call(
        _affine_kernel,
        grid=(N_GROUPS, NB),
        in_specs=[
            pl.BlockSpec((1, BLK, C_IN), lambda g, n: (g, n, 0)),
            pl.BlockSpec((1, C_OUT, C_IN), lambda g, n: (g, 0, 0)),
            pl.BlockSpec((1, 1, C_OUT), lambda g, n: (g, 0, 0)),
        ],
        out_specs=pl.BlockSpec((BLK, C_OUT), lambda g, n: (g * NB + n, 0)),
        out_shape=jax.ShapeDtypeStruct((N_GROUPS * N_PER_GROUP, C_OUT), jnp.float32),
    )(inputs, weights, bias3)
    return out
